# baseline (speedup 1.0000x reference)
import functools

import jax
import jax.numpy as jnp
from jax import lax
from jax.experimental import pallas as pl
from jax.experimental.pallas import tpu as pltpu
from jax.experimental.pallas import tpu_sc as plsc

MU_C = 1000
BATCH_C = 1024
SEQ_C = 50

_INFO = plsc.get_sparse_core_info()
_NC = _INFO.num_cores
_NS = _INFO.num_subcores
_NW = _NC * _NS
_L = _INFO.num_lanes

_B_PER_W = BATCH_C // _NW
_ROW = MU_C * SEQ_C
_XCHUNK = _B_PER_W * SEQ_C
_NVEC = (SEQ_C + _L - 1) // _L


def _make_sc_call():
    mesh = plsc.VectorSubcoreMesh(core_axis_name="c", subcore_axis_name="s")

    @functools.partial(
        pl.kernel,
        mesh=mesh,
        out_type=jax.ShapeDtypeStruct((BATCH_C, _ROW), jnp.float32),
        scratch_types=[
            pltpu.VMEM((_XCHUNK + _L,), jnp.int32),
            pltpu.VMEM((_ROW,), jnp.float32),
            pltpu.VMEM((_ROW,), jnp.float32),
            pltpu.SemaphoreType.DMA,
            pltpu.SemaphoreType.DMA,
        ],
        compiler_params=pltpu.CompilerParams(needs_layout_passes=False),
    )
    def one_hot_sc(x_hbm, out_hbm, x_v, buf0, buf1, sem0, sem1):
        wid = lax.axis_index("s") * _NC + lax.axis_index("c")
        base = wid * _B_PER_W

        pltpu.sync_copy(
            x_hbm.at[pl.ds(base * SEQ_C, _XCHUNK)], x_v.at[pl.ds(0, _XCHUNK)]
        )

        lanes = lax.iota(jnp.int32, _L)
        fzero = jnp.zeros((_L,), jnp.float32)
        fone = jnp.ones((_L,), jnp.float32)

        def zero_body(i, carry):
            k = i * (_L * 25)
            for u in range(25):
                buf0[pl.ds(k + u * _L, _L)] = fzero
                buf1[pl.ds(k + u * _L, _L)] = fzero
            return carry

        lax.fori_loop(0, _ROW // (_L * 25), zero_body, 0)

        def offsets(b_local, j):
            s0 = b_local * SEQ_C + j * _L
            mask = None
            if (j + 1) * _L > SEQ_C:
                mask = lanes < (SEQ_C - j * _L)
            xv = x_v[pl.ds(s0, _L)]
            off = xv * SEQ_C + (lanes + j * _L)
            return off, mask

        bufs = (buf0, buf1)
        sems = (sem0, sem1)
        pending = [None, None]

        for b_local in range(_B_PER_W):
            slot = b_local % 2
            buf = bufs[slot]
            if pending[slot] is not None:
                pending[slot].wait()
                for j in range(_NVEC):
                    off, mask = offsets(b_local - 2, j)
                    plsc.store_scatter(buf, [off], fzero, mask=mask)
            for j in range(_NVEC):
                off, mask = offsets(b_local, j)
                plsc.store_scatter(buf, [off], fone, mask=mask)
            pending[slot] = pltpu.async_copy(
                buf, out_hbm.at[base + b_local], sems[slot]
            )

        pending[0].wait()
        pending[1].wait()

    return one_hot_sc


_sc_call = _make_sc_call()


@jax.jit
def kernel(x, ones):
    del ones
    x_flat = x.reshape(-1).astype(jnp.int32)
    out = _sc_call(x_flat)
    return out.reshape(BATCH_C, MU_C, SEQ_C)


# speedup vs baseline: 1.5402x; 1.5402x over previous
import functools

import jax
import jax.numpy as jnp
from jax import lax
from jax.experimental import pallas as pl
from jax.experimental.pallas import tpu as pltpu
from jax.experimental.pallas import tpu_sc as plsc

MU_C = 1000
BATCH_C = 1024
SEQ_C = 50
LANE_PAD = 128
HALF_MU = MU_C // 2

_INFO = plsc.get_sparse_core_info()
_NC = _INFO.num_cores
_NS = _INFO.num_subcores
_NW = _NC * _NS
_L = _INFO.num_lanes

_B_PER_W = BATCH_C // _NW
_PIECE = HALF_MU * LANE_PAD
_XCHUNK = _B_PER_W * SEQ_C
_NVEC = (SEQ_C + _L - 1) // _L


def _make_sc_call():
    mesh = plsc.VectorSubcoreMesh(core_axis_name="c", subcore_axis_name="s")

    @functools.partial(
        pl.kernel,
        mesh=mesh,
        out_type=jax.ShapeDtypeStruct((2 * BATCH_C, _PIECE), jnp.float32),
        scratch_types=[
            pltpu.VMEM((_XCHUNK + _L,), jnp.int32),
            pltpu.VMEM((_PIECE,), jnp.float32),
            pltpu.VMEM((_PIECE,), jnp.float32),
            pltpu.SemaphoreType.DMA,
            pltpu.SemaphoreType.DMA,
        ],
        compiler_params=pltpu.CompilerParams(needs_layout_passes=False),
    )
    def one_hot_sc(x_hbm, out_hbm, x_v, buf0, buf1, sem0, sem1):
        wid = lax.axis_index("s") * _NC + lax.axis_index("c")
        base = wid * _B_PER_W

        pltpu.sync_copy(
            x_hbm.at[pl.ds(base * SEQ_C, _XCHUNK)], x_v.at[pl.ds(0, _XCHUNK)]
        )

        lanes = lax.iota(jnp.int32, _L)
        fzero = jnp.zeros((_L,), jnp.float32)
        fone = jnp.ones((_L,), jnp.float32)

        def zero_body(i, carry):
            k = i * (_L * 25)
            for u in range(25):
                buf0[pl.ds(k + u * _L, _L)] = fzero
                buf1[pl.ds(k + u * _L, _L)] = fzero
            return carry

        lax.fori_loop(0, _PIECE // (_L * 25), zero_body, 0)

        def load_batch(b_local):
            vecs = []
            for j in range(_NVEC):
                s0 = b_local * SEQ_C + j * _L
                valid = None
                if (j + 1) * _L > SEQ_C:
                    valid = lanes < (SEQ_C - j * _L)
                vecs.append((x_v[pl.ds(s0, _L)], lanes + j * _L, valid))
            return vecs

        def scatter(buf, vecs, half, value):
            for xv, s_idx, valid in vecs:
                if half == 0:
                    m = xv < HALF_MU
                else:
                    m = xv >= HALF_MU
                if valid is not None:
                    m = jnp.logical_and(m, valid)
                off = (xv - half * HALF_MU) * LANE_PAD + s_idx
                plsc.store_scatter(buf, [off], value, mask=m)

        bufs = (buf0, buf1)
        sems = (sem0, sem1)
        pending = [None, None]
        prev_vecs = None

        for b_local in range(_B_PER_W):
            vecs = load_batch(b_local)
            for half in range(2):
                buf = bufs[half]
                if pending[half] is not None:
                    pending[half].wait()
                    scatter(buf, prev_vecs, half, fzero)
                scatter(buf, vecs, half, fone)
                pending[half] = pltpu.async_copy(
                    buf, out_hbm.at[(base + b_local) * 2 + half], sems[half]
                )
            prev_vecs = vecs

        pending[0].wait()
        pending[1].wait()

    return one_hot_sc


_sc_call = _make_sc_call()


@jax.jit
def kernel(x, ones):
    del ones
    x_flat = x.reshape(-1).astype(jnp.int32)
    out = _sc_call(x_flat)
    out = out.reshape(BATCH_C, MU_C, LANE_PAD)
    return lax.slice(out, (0, 0, 0), (BATCH_C, MU_C, SEQ_C))


# speedup vs baseline: 2.0944x; 1.3599x over previous
import functools

import jax
import jax.numpy as jnp
from jax import lax
from jax.experimental import pallas as pl
from jax.experimental.pallas import tpu as pltpu
from jax.experimental.pallas import tpu_sc as plsc

MU_C = 1000
BATCH_C = 1024
SEQ_C = 50
ROW_SPLIT = 512

_INFO = plsc.get_sparse_core_info()
_NC = _INFO.num_cores
_NS = _INFO.num_subcores
_NW = _NC * _NS
_L = _INFO.num_lanes

_B_PER_W = BATCH_C // _NW
_XCHUNK = _B_PER_W * SEQ_C
_NVEC = (SEQ_C + _L - 1) // _L
_PIECE_ROWS = (ROW_SPLIT, MU_C - ROW_SPLIT)


def _make_sc_call():
    mesh = plsc.VectorSubcoreMesh(core_axis_name="c", subcore_axis_name="s")

    @functools.partial(
        pl.kernel,
        mesh=mesh,
        out_type=jax.ShapeDtypeStruct((BATCH_C, MU_C, SEQ_C), jnp.float32),
        scratch_types=[
            pltpu.VMEM((_XCHUNK + _L,), jnp.int32),
            pltpu.VMEM((_PIECE_ROWS[0], SEQ_C), jnp.float32),
            pltpu.VMEM((_PIECE_ROWS[1], SEQ_C), jnp.float32),
            pltpu.SemaphoreType.DMA,
            pltpu.SemaphoreType.DMA,
        ],
        compiler_params=pltpu.CompilerParams(needs_layout_passes=False),
    )
    def one_hot_sc(x_hbm, out_hbm, x_v, buf0, buf1, sem0, sem1):
        wid = lax.axis_index("s") * _NC + lax.axis_index("c")
        base = wid * _B_PER_W

        pltpu.sync_copy(
            x_hbm.at[pl.ds(base * SEQ_C, _XCHUNK)], x_v.at[pl.ds(0, _XCHUNK)]
        )

        lanes = lax.iota(jnp.int32, _L)
        fzero = jnp.zeros((_L,), jnp.float32)
        fone = jnp.ones((_L,), jnp.float32)

        col_tail_mask = lanes < (SEQ_C - 3 * _L)

        def zero_row(buf, r):
            row = jnp.full((_L,), r, jnp.int32)
            for j in range(_NVEC):
                cols = lanes + j * _L
                m = col_tail_mask if j == _NVEC - 1 else None
                plsc.store_scatter(buf, [row, cols], fzero, mask=m)

        def zero_both(r, carry):
            zero_row(buf0, r)
            zero_row(buf1, r)
            return carry

        def zero_b0(r, carry):
            zero_row(buf0, r)
            return carry

        lax.fori_loop(0, _PIECE_ROWS[1], zero_both, 0, unroll=2)
        lax.fori_loop(_PIECE_ROWS[1], _PIECE_ROWS[0], zero_b0, 0, unroll=2)

        def load_batch(b_local):
            vecs = []
            for j in range(_NVEC):
                s0 = b_local * SEQ_C + j * _L
                valid = None
                if (j + 1) * _L > SEQ_C:
                    valid = lanes < (SEQ_C - j * _L)
                vecs.append((x_v[pl.ds(s0, _L)], lanes + j * _L, valid))
            return vecs

        def scatter(buf, vecs, piece, value):
            for xv, s_idx, valid in vecs:
                if piece == 0:
                    m = xv < ROW_SPLIT
                    row = xv
                else:
                    m = xv >= ROW_SPLIT
                    row = xv - ROW_SPLIT
                if valid is not None:
                    m = jnp.logical_and(m, valid)
                plsc.store_scatter(buf, [row, s_idx], value, mask=m)

        bufs = (buf0, buf1)
        sems = (sem0, sem1)
        pending = [None, None]
        prev_vecs = None

        for b_local in range(_B_PER_W):
            vecs = load_batch(b_local)
            for piece in range(2):
                buf = bufs[piece]
                if pending[piece] is not None:
                    pending[piece].wait()
                    scatter(buf, prev_vecs, piece, fzero)
                scatter(buf, vecs, piece, fone)
                row0 = piece * ROW_SPLIT
                pending[piece] = pltpu.async_copy(
                    buf,
                    out_hbm.at[base + b_local, pl.ds(row0, _PIECE_ROWS[piece])],
                    sems[piece],
                )
            prev_vecs = vecs

        pending[0].wait()
        pending[1].wait()

    return one_hot_sc


_sc_call = _make_sc_call()


@jax.jit
def kernel(x, ones):
    del ones
    x_flat = x.reshape(-1).astype(jnp.int32)
    return _sc_call(x_flat)


# speedup vs baseline: 11.9670x; 5.7138x over previous
import functools

import jax
import jax.numpy as jnp
from jax import lax
from jax.experimental import pallas as pl
from jax.experimental.pallas import tpu as pltpu
from jax.experimental.pallas import tpu_sc as plsc

MU_C = 1000
BATCH_C = 1024
SEQ_C = 50
M_PIECE = 40
PPS = MU_C // M_PIECE
NPIECES = SEQ_C * PPS

_INFO = plsc.get_sparse_core_info()
_NC = _INFO.num_cores
_NS = _INFO.num_subcores
_NW = _NC * _NS
_L = _INFO.num_lanes

_K_MAX = (NPIECES + _NW - 1) // _NW
_NBVEC = BATCH_C // _L


def _make_sc_call():
    mesh = plsc.VectorSubcoreMesh(core_axis_name="c", subcore_axis_name="s")

    @functools.partial(
        pl.kernel,
        mesh=mesh,
        out_type=jax.ShapeDtypeStruct((SEQ_C, MU_C, BATCH_C), jnp.float32),
        scratch_types=[
            pltpu.VMEM((_K_MAX * BATCH_C,), jnp.int32),
            pltpu.VMEM((M_PIECE, BATCH_C), jnp.float32),
            pltpu.VMEM((M_PIECE, BATCH_C), jnp.float32),
            pltpu.SemaphoreType.DMA,
            pltpu.SemaphoreType.DMA,
            pltpu.SemaphoreType.DMA,
        ],
        compiler_params=pltpu.CompilerParams(needs_layout_passes=False),
    )
    def one_hot_sc(xt_hbm, out_hbm, x_all, buf0, buf1, semx, sem0, sem1):
        wid = lax.axis_index("s") * _NC + lax.axis_index("c")

        lanes = lax.iota(jnp.int32, _L)
        fzero = jnp.zeros((_L,), jnp.float32)
        fone = jnp.ones((_L,), jnp.float32)

        def piece_params(k):
            p = wid + _NW * k
            return p, p // PPS, (p % PPS) * M_PIECE

        def stage(k, carry):
            p, s, _ = piece_params(k)

            @pl.when(p < NPIECES)
            def _():
                pltpu.async_copy(
                    xt_hbm.at[s], x_all.at[pl.ds(k * BATCH_C, BATCH_C)], semx
                )

            return carry

        lax.fori_loop(0, _K_MAX, stage, 0)

        def zero_row(r, carry):
            row = jnp.full((_L,), r, jnp.int32)
            for v in range(_NBVEC):
                cols = lanes + v * _L
                plsc.store_scatter(buf0, [row, cols], fzero)
                plsc.store_scatter(buf1, [row, cols], fzero)
            return carry

        lax.fori_loop(0, M_PIECE, zero_row, 0)

        def drain(k, carry):
            p, _, _ = piece_params(k)

            @pl.when(p < NPIECES)
            def _():
                pltpu.make_async_copy(
                    xt_hbm.at[0], x_all.at[pl.ds(0, BATCH_C)], semx
                ).wait()

            return carry

        lax.fori_loop(0, _K_MAX, drain, 0)

        def scan_scatter(buf, k, m0, value):
            for v in range(_NBVEC):
                xv = x_all[pl.ds(k * BATCH_C + v * _L, _L)]
                m = jnp.logical_and(xv >= m0, xv < m0 + M_PIECE)
                row = xv - m0
                cols = lanes + v * _L
                plsc.store_scatter(buf, [row, cols], value, mask=m)

        def do_piece(k2, k, buf, sem):
            p, s, m0 = piece_params(k)

            @pl.when(k2 > 0)
            def _():
                pltpu.make_async_copy(
                    buf, out_hbm.at[0, pl.ds(0, M_PIECE)], sem
                ).wait()
                _, _, m0p = piece_params(k - 2)
                scan_scatter(buf, k - 2, m0p, fzero)

            scan_scatter(buf, k, m0, fone)
            pltpu.async_copy(buf, out_hbm.at[s, pl.ds(m0, M_PIECE)], sem)

        def main_body(k2, carry):
            do_piece(k2, 2 * k2, buf0, sem0)
            p1 = wid + _NW * (2 * k2 + 1)

            @pl.when(p1 < NPIECES)
            def _():
                do_piece(k2, 2 * k2 + 1, buf1, sem1)

            return carry

        lax.fori_loop(0, _K_MAX // 2, main_body, 0)

        pltpu.make_async_copy(buf0, out_hbm.at[0, pl.ds(0, M_PIECE)], sem0).wait()
        pltpu.make_async_copy(buf1, out_hbm.at[0, pl.ds(0, M_PIECE)], sem1).wait()

    return one_hot_sc


_sc_call = _make_sc_call()


@jax.jit
def kernel(x, ones):
    del ones
    xt = jnp.transpose(x.astype(jnp.int32), (1, 0))
    out_smb = _sc_call(xt)
    return jnp.transpose(out_smb, (2, 1, 0))
